# Initial kernel scaffold; baseline (speedup 1.0000x reference)
import jax
import jax.numpy as jnp
from jax.experimental import pallas as pl


def kernel(x, Wq, Wk, Wv, Wo, Wdq, Wdk):
    raise NotImplementedError("write your pallas kernel here")


# speedup vs baseline: 2.7423x; 2.7423x over previous
import functools

import jax
import jax.numpy as jnp
from jax.experimental import pallas as pl
from jax.experimental.pallas import tpu as pltpu

B, T, C = 1, 2048, 768
H, D = 12, 64
K, P = 4, 2
ALPHA, BETA = 2.0, 32.0

TQ = 256


def _proj_kernel(x_ref, xT_ref, wqT_ref, wk_ref, wvT_ref, wdqT_ref, wdk_ref,
                 dq_ref, dkT_ref, v_ref):
    x = x_ref[...]
    xT = xT_ref[...]
    qh = jnp.dot(x, wqT_ref[0], preferred_element_type=jnp.float32)
    kTh = jnp.dot(wk_ref[0], xT, preferred_element_type=jnp.float32)
    scale = jnp.float32(P - 1)
    dq_ref[0] = jax.nn.sigmoid(
        jnp.dot(qh, wdqT_ref[...], preferred_element_type=jnp.float32)) * scale
    dkT_ref[0] = jax.nn.sigmoid(
        jnp.dot(wdk_ref[...], kTh, preferred_element_type=jnp.float32)) * scale
    v_ref[0] = jnp.dot(x, wvT_ref[0], preferred_element_type=jnp.float32)


def _lcp_weights(dq, dkT):
    cum = None
    lcp = None
    for l in range(K):
        a = dq[:, l:l + 1]
        b = dkT[l:l + 1, :]
        diff = jnp.abs(a - b)
        m = jax.nn.sigmoid(BETA * (jnp.float32(0.5) - diff))
        cum = m if cum is None else cum * m
        lcp = cum if lcp is None else lcp + cum
    return jnp.exp2(lcp)


def _attn_kernel(dq_ref, dkT_ref, v_ref, woT_ref, y_ref):
    i = pl.program_id(0)
    h = pl.program_id(1)
    dq = dq_ref[0]

    def body(j, carry):
        acc, den = carry
        dkT = dkT_ref[0, :, pl.ds(j * TQ, TQ)]
        vblk = v_ref[0, pl.ds(j * TQ, TQ), :]
        w = _lcp_weights(dq, dkT)
        acc = acc + jnp.dot(w, vblk, preferred_element_type=jnp.float32)
        den = den + jnp.sum(w, axis=1, keepdims=True)
        return acc, den

    acc0 = jnp.zeros((TQ, D), jnp.float32)
    den0 = jnp.zeros((TQ, 1), jnp.float32)
    acc, den = jax.lax.fori_loop(0, i, body, (acc0, den0))

    dkT = dkT_ref[0, :, pl.ds(i * TQ, TQ)]
    vblk = v_ref[0, pl.ds(i * TQ, TQ), :]
    w = _lcp_weights(dq, dkT)
    rows = jax.lax.broadcasted_iota(jnp.int32, (TQ, TQ), 0)
    cols = jax.lax.broadcasted_iota(jnp.int32, (TQ, TQ), 1)
    w = jnp.where(cols <= rows, w, jnp.float32(0.0))
    acc = acc + jnp.dot(w, vblk, preferred_element_type=jnp.float32)
    den = den + jnp.sum(w, axis=1, keepdims=True)

    out = acc / den
    y = jnp.dot(out, woT_ref[...], preferred_element_type=jnp.float32)

    @pl.when(h == 0)
    def _():
        y_ref[...] = y

    @pl.when(h > 0)
    def _():
        y_ref[...] = y_ref[...] + y


@jax.jit
def _forward(x, Wq, Wk, Wv, Wo, Wdq, Wdk):
    x2 = x.reshape(T, C)
    dq, dkT, v = pl.pallas_call(
        _proj_kernel,
        grid=(H,),
        in_specs=[
            pl.BlockSpec((T, C), lambda h: (0, 0)),
            pl.BlockSpec((C, T), lambda h: (0, 0)),
            pl.BlockSpec((1, C, D), lambda h: (h, 0, 0)),
            pl.BlockSpec((1, D, C), lambda h: (h, 0, 0)),
            pl.BlockSpec((1, C, D), lambda h: (h, 0, 0)),
            pl.BlockSpec((D, K), lambda h: (0, 0)),
            pl.BlockSpec((K, D), lambda h: (0, 0)),
        ],
        out_specs=(
            pl.BlockSpec((1, T, K), lambda h: (h, 0, 0)),
            pl.BlockSpec((1, K, T), lambda h: (h, 0, 0)),
            pl.BlockSpec((1, T, D), lambda h: (h, 0, 0)),
        ),
        out_shape=(
            jax.ShapeDtypeStruct((H, T, K), jnp.float32),
            jax.ShapeDtypeStruct((H, K, T), jnp.float32),
            jax.ShapeDtypeStruct((H, T, D), jnp.float32),
        ),
        compiler_params=pltpu.CompilerParams(
            dimension_semantics=("arbitrary",),
        ),
    )(x2, x2.T,
      Wq.T.reshape(C, H, D).transpose(1, 0, 2),
      Wk.reshape(H, D, C),
      Wv.T.reshape(C, H, D).transpose(1, 0, 2),
      Wdq.T, Wdk)

    nq = T // TQ
    y = pl.pallas_call(
        _attn_kernel,
        grid=(nq, H),
        in_specs=[
            pl.BlockSpec((1, TQ, K), lambda i, h: (h, i, 0)),
            pl.BlockSpec((1, K, T), lambda i, h: (h, 0, 0)),
            pl.BlockSpec((1, T, D), lambda i, h: (h, 0, 0)),
            pl.BlockSpec((D, C), lambda i, h: (h, 0)),
        ],
        out_specs=pl.BlockSpec((TQ, C), lambda i, h: (i, 0)),
        out_shape=jax.ShapeDtypeStruct((T, C), jnp.float32),
        compiler_params=pltpu.CompilerParams(
            dimension_semantics=("arbitrary", "arbitrary"),
        ),
    )(dq, dkT, v, Wo.T)
    return y.reshape(B, T, C)


def kernel(x, Wq, Wk, Wv, Wo, Wdq, Wdk):
    return _forward(x, Wq, Wk, Wv, Wo, Wdq, Wdk)
